# baseline (speedup 1.0000x reference)
import jax
import jax.numpy as jnp
from jax import lax
from jax.experimental import pallas as pl
from jax.experimental.pallas import tpu as pltpu, tpu_sc as plsc

_N_VAL = 1000
_NR, _NC = 4096, 20
_NKB = 8
_PC = 256
_NW = 32
_IPW = _NR // _NW
_VPW = _IPW * _NC

_EB = 16


def _sc_body(x_hbm, zeros_hbm, pos_hbm, idx_v, slab_v):
    wid = lax.axis_index("s") * 2 + lax.axis_index("c")
    base_i = wid * _IPW

    pltpu.sync_copy(x_hbm.at[pl.ds(base_i * _NC, _VPW)], idx_v)
    pltpu.sync_copy(zeros_hbm, slab_v)

    lane = lax.iota(jnp.int32, 16)

    def scat(v, carry):
        flat = v * 16 + lane
        xv = idx_v[pl.ds(v * 16, 16)]
        row = flat // _NC
        col = lax.shift_right_logical(xv, 7) * _NC + (flat - row * _NC)
        val = lax.bitwise_and(xv, 127) + 1
        plsc.store_scatter(slab_v, [row, col], val)
        return carry
    lax.fori_loop(0, _VPW // 16, scat, 0)

    pltpu.sync_copy(slab_v, pos_hbm.at[pl.ds(base_i, _IPW)])


def _expand_body(pos_ref, o_ref):
    for kb in range(_NKB):
        kw = min(128, _N_VAL - kb * 128)
        iota1 = lax.broadcasted_iota(jnp.int32, (_EB, _NC, kw), 2) + 1
        wb = pos_ref[:, pl.ds(kb * _NC, _NC)]
        v = jnp.where(wb[:, :, None] == iota1, 1.0, 0.0)
        o_ref[:, :, pl.ds(kb * 128, kw)] = v.astype(jnp.float32)


def kernel(x):
    xf = x.reshape(-1)
    zeros = jnp.zeros((_IPW, _PC), jnp.int32)
    mesh = plsc.VectorSubcoreMesh(core_axis_name="c", subcore_axis_name="s")
    posidx = pl.kernel(
        _sc_body,
        out_type=jax.ShapeDtypeStruct((_NR, _PC), jnp.int32),
        mesh=mesh,
        scratch_types=[
            pltpu.VMEM((_VPW,), jnp.int32),
            pltpu.VMEM((_IPW, _PC), jnp.int32),
        ],
        compiler_params=pltpu.CompilerParams(
            needs_layout_passes=False, use_tc_tiling_on_sc=True
        ),
    )(xf, zeros)
    return pl.pallas_call(
        _expand_body,
        grid=(_NR // _EB,),
        in_specs=[pl.BlockSpec((_EB, _PC), lambda i: (i, 0))],
        out_specs=pl.BlockSpec((_EB, _NC, _N_VAL), lambda i: (i, 0, 0)),
        out_shape=jax.ShapeDtypeStruct((_NR, _NC, _N_VAL), jnp.float32),
    )(posidx)


# speedup vs baseline: 1.1804x; 1.1804x over previous
import jax
import jax.numpy as jnp
from jax import lax
from jax.experimental import pallas as pl
from jax.experimental.pallas import tpu as pltpu, tpu_sc as plsc

_N_VAL = 1000
_NR, _NC = 4096, 20
_NKB = 8
_PC = 256
_NW = 32
_IPW = _NR // _NW
_VPW = _IPW * _NC

_EB = 64


def _sc_body(x_hbm, zeros_hbm, pos_hbm, idx_v, slab_v):
    wid = lax.axis_index("s") * 2 + lax.axis_index("c")
    base_i = wid * _IPW

    pltpu.sync_copy(x_hbm.at[pl.ds(base_i * _NC, _VPW)], idx_v)
    pltpu.sync_copy(zeros_hbm, slab_v)

    lane = lax.iota(jnp.int32, 16)

    def scat(v, carry):
        flat = v * 16 + lane
        xv = idx_v[pl.ds(v * 16, 16)]
        row = flat // _NC
        col = lax.shift_right_logical(xv, 7) * _NC + (flat - row * _NC)
        val = lax.bitwise_and(xv, 127) + 1
        plsc.store_scatter(slab_v, [row, col], val)
        return carry
    lax.fori_loop(0, _VPW // 16, scat, 0)

    pltpu.sync_copy(slab_v, pos_hbm.at[pl.ds(base_i, _IPW)])


def _expand_body(pos_ref, o_ref):
    for kb in range(_NKB):
        kw = min(128, _N_VAL - kb * 128)
        iota1 = lax.broadcasted_iota(jnp.int32, (_EB, _NC, kw), 2) + 1
        wb = pos_ref[:, pl.ds(kb * _NC, _NC)]
        v = jnp.where(wb[:, :, None] == iota1, 1.0, 0.0)
        o_ref[:, :, pl.ds(kb * 128, kw)] = v.astype(jnp.float32)


def kernel(x):
    xf = x.reshape(-1)
    zeros = jnp.zeros((_IPW, _PC), jnp.int32)
    mesh = plsc.VectorSubcoreMesh(core_axis_name="c", subcore_axis_name="s")
    posidx = pl.kernel(
        _sc_body,
        out_type=jax.ShapeDtypeStruct((_NR, _PC), jnp.int32),
        mesh=mesh,
        scratch_types=[
            pltpu.VMEM((_VPW,), jnp.int32),
            pltpu.VMEM((_IPW, _PC), jnp.int32),
        ],
        compiler_params=pltpu.CompilerParams(
            needs_layout_passes=False, use_tc_tiling_on_sc=True
        ),
    )(xf, zeros)
    return pl.pallas_call(
        _expand_body,
        grid=(_NR // _EB,),
        in_specs=[pl.BlockSpec((_EB, _PC), lambda i: (i, 0))],
        out_specs=pl.BlockSpec((_EB, _NC, _N_VAL), lambda i: (i, 0, 0)),
        out_shape=jax.ShapeDtypeStruct((_NR, _NC, _N_VAL), jnp.float32),
    )(posidx)


# speedup vs baseline: 1.4704x; 1.2457x over previous
import jax
import jax.numpy as jnp
from jax import lax
from jax.experimental import pallas as pl
from jax.experimental.pallas import tpu as pltpu, tpu_sc as plsc

_N_VAL = 1000
_NR, _NC = 4096, 20
_NKB = 8
_PC = 256
_NW = 32
_IPW = _NR // _NW
_VPW = _IPW * _NC

_EB = 64


def _sc_body(x_hbm, zeros_hbm, pos_hbm, idx_v, slab_v):
    wid = lax.axis_index("s") * 2 + lax.axis_index("c")
    base_i = wid * _IPW

    pltpu.sync_copy(x_hbm.at[pl.ds(base_i * _NC, _VPW)], idx_v)
    pltpu.sync_copy(zeros_hbm, slab_v)

    lane = lax.iota(jnp.int32, 16)

    def scat(v, carry):
        flat = v * 16 + lane
        xv = idx_v[pl.ds(v * 16, 16)]
        row = flat // _NC
        col = lax.shift_right_logical(xv, 7) * 24 + (flat - row * _NC)
        val = lax.bitwise_and(xv, 127) + 1
        plsc.store_scatter(slab_v, [row, col], val)
        return carry
    lax.fori_loop(0, _VPW // 16, scat, 0)

    pltpu.sync_copy(slab_v, pos_hbm.at[pl.ds(base_i, _IPW)])


def _expand_body(pos_ref, o_ref):
    iota1 = lax.broadcasted_iota(jnp.int32, (_EB, 24, 128), 2) + 1
    for kb in range(_NKB):
        wb = pos_ref[:, pl.ds(kb * 24, 24)]
        v = jnp.where(wb[:, :, None] == iota1, 1.0, 0.0)
        o_ref[:, :, pl.ds(kb * 128, 128)] = v.astype(jnp.float32)


def kernel(x):
    xf = x.reshape(-1)
    zeros = jnp.zeros((_IPW, _PC), jnp.int32)
    mesh = plsc.VectorSubcoreMesh(core_axis_name="c", subcore_axis_name="s")
    posidx = pl.kernel(
        _sc_body,
        out_type=jax.ShapeDtypeStruct((_NR, _PC), jnp.int32),
        mesh=mesh,
        scratch_types=[
            pltpu.VMEM((_VPW,), jnp.int32),
            pltpu.VMEM((_IPW, _PC), jnp.int32),
        ],
        compiler_params=pltpu.CompilerParams(
            needs_layout_passes=False, use_tc_tiling_on_sc=True
        ),
    )(xf, zeros)
    full = pl.pallas_call(
        _expand_body,
        grid=(_NR // _EB,),
        in_specs=[pl.BlockSpec((_EB, _PC), lambda i: (i, 0))],
        out_specs=pl.BlockSpec((_EB, 24, 1024), lambda i: (i, 0, 0)),
        out_shape=jax.ShapeDtypeStruct((_NR, 24, 1024), jnp.float32),
    )(posidx)
    return full[:, :_NC, :_N_VAL]
